# Initial kernel scaffold; baseline (speedup 1.0000x reference)
import jax
import jax.numpy as jnp
from jax.experimental import pallas as pl


def kernel(x, emotion_tags, emb_table, ln_gamma, ln_beta):
    raise NotImplementedError("write your pallas kernel here")


# speedup vs baseline: 4.6280x; 4.6280x over previous
import jax
import jax.numpy as jnp
from jax import lax
from jax.experimental import pallas as pl
from jax.experimental.pallas import tpu as pltpu

EPS = 1e-12


def _tc_body(tags_ref, x_ref, emb_ref, gamma_ref, beta_ref, out_ref):
    x = x_ref[...]
    sel = tags_ref[...] != 0
    t0 = emb_ref[0, :][None, :]
    t1 = emb_ref[1, :][None, :]
    h = x + jnp.where(sel, t1, t0)
    mean = jnp.mean(h, axis=-1, keepdims=True)
    var = jnp.mean(jnp.square(h - mean), axis=-1, keepdims=True)
    rstd = lax.rsqrt(var + EPS)
    g = gamma_ref[0, :][None, :]
    b = beta_ref[0, :][None, :]
    out_ref[...] = (h - mean) * rstd * g + b


def kernel(x, emotion_tags, emb_table, ln_gamma, ln_beta):
    B, L, D = x.shape
    N = B * L
    assert D == 128
    RB = 2048
    while N % RB:
        RB //= 2
    x2 = x.reshape(N, D)
    tagsc = emotion_tags.astype(jnp.int32).reshape(N, 1)
    g2 = ln_gamma.reshape(1, D)
    b2 = ln_beta.reshape(1, D)

    out = pl.pallas_call(
        _tc_body,
        grid=(N // RB,),
        in_specs=[
            pl.BlockSpec((RB, 1), lambda i: (i, 0)),
            pl.BlockSpec((RB, D), lambda i: (i, 0)),
            pl.BlockSpec((2, D), lambda i: (0, 0)),
            pl.BlockSpec((1, D), lambda i: (0, 0)),
            pl.BlockSpec((1, D), lambda i: (0, 0)),
        ],
        out_specs=pl.BlockSpec((RB, D), lambda i: (i, 0)),
        out_shape=jax.ShapeDtypeStruct((N, D), jnp.float32),
    )(tagsc, x2, emb_table, g2, b2)
    return out.reshape(B, L, D)
